# baseline (speedup 1.0000x reference)
import functools

import jax
import jax.numpy as jnp
from jax import lax
from jax.experimental import pallas as pl
from jax.experimental.pallas import tpu as pltpu
from jax.experimental.pallas import tpu_sc as plsc

_N_HEADS = 26
_DIM = 32
_HEAD_ROWS = 100000
_WIN = 100096
_HWIN = _WIN // 2
_TAIL = 64
_CHUNK = 128
_CSZ = [12416, 12544, 12544, 12544]
_COFF = [0, 12416, 24960, 37504]
_CBUF = max(_CSZ)


def _make_kernel(batch):
    mesh = plsc.VectorSubcoreMesh(core_axis_name="c", subcore_axis_name="s")
    half = batch // 2
    n_ct = half // _CHUNK

    @functools.partial(
        pl.kernel,
        mesh=mesh,
        compiler_params=pltpu.CompilerParams(use_tc_tiling_on_sc=True),
        out_type=jax.ShapeDtypeStruct(
            (_N_HEADS, _DIM // 8, batch // 128, 8, 128), jnp.float32),
        scratch_types=[
            pltpu.VMEM_SHARED((8 * (_WIN + _TAIL),), jnp.float32),
            pltpu.VMEM((2 * _CBUF + _TAIL,), jnp.float32),
            pltpu.VMEM((half,), jnp.int32),
            pltpu.VMEM((n_ct, _CHUNK), jnp.float32),
            pltpu.VMEM((_CSZ[0],), jnp.int32),
            pltpu.VMEM((_CBUF,), jnp.int32),
            pltpu.SemaphoreType.DMA,
            pltpu.SemaphoreType.DMA,
        ],
    )
    def emb(tab_hbm, ids_hbm, tail_hbm, iota_hbm, out_hbm, win_sh, stage_v,
            ids_v, res_v, iota_a, iota_b, sem, psem):
        core = lax.axis_index("c")
        sub = lax.axis_index("s")
        floc = sub // 2
        p = sub % 2
        fs = floc % 8
        wbase = floc * (_WIN + _TAIL)

        pltpu.sync_copy(iota_hbm.at[pl.ds(0, _CSZ[0])], iota_a)
        pltpu.sync_copy(iota_hbm, iota_b)

        def pass_body(pa, carry0):
            f = core * 16 + pa * 8 + floc
            fr = core * 2 + pa

            pltpu.sync_copy(tail_hbm.at[pl.ds(f * _TAIL, _TAIL)],
                            stage_v.at[pl.ds(2 * _CBUF, _TAIL)])

            @pl.when(p == 1)
            def _publish_tail():
                pltpu.sync_copy(stage_v.at[pl.ds(2 * _CBUF, _TAIL)],
                                win_sh.at[pl.ds(wbase + _WIN, _TAIL)])

            def head_body(h, carry):
                off_h = h * _HEAD_ROWS
                c0 = pl.multiple_of(
                    (off_h & ~127) - (h // (_N_HEADS - 1)) * 128, 128)
                shift = off_h - c0

                cc = pl.multiple_of(c0 + p * _HWIN, 128)
                pub0 = wbase + p * _HWIN
                handles = []
                for k in range(len(_CSZ)):
                    if k >= 2:
                        handles[k - 2].wait()
                    buf = stage_v.at[pl.ds((k % 2) * _CBUF, _CSZ[k])]
                    ck = pl.multiple_of(cc + _COFF[k], 128)
                    for s in range(8):
                        @pl.when(fs == s)
                        def _copy_win(s=s, buf=buf, ck=ck, k=k):
                            pltpu.sync_copy(
                                tab_hbm.at[fr, s, pl.ds(ck, _CSZ[k])], buf)
                    iot = iota_a if k == 0 else iota_b
                    handles.append(pltpu.async_copy(
                        buf,
                        win_sh.at[pl.ds(pub0 + _COFF[k], _CSZ[k])].at[iot],
                        psem))
                handles[-2].wait()
                handles[-1].wait()

                plsc.subcore_barrier()

                win_view = win_sh.at[pl.ds(wbase + shift, _HEAD_ROWS)]
                b0 = p * half
                pltpu.sync_copy(ids_hbm.at[pl.ds(h * batch + b0, half)],
                                ids_v)
                copies = []
                for j in range(n_ct):
                    copies.append(pltpu.async_copy(
                        win_view.at[ids_v.at[pl.ds(j * _CHUNK, _CHUNK)]],
                        res_v.at[j], sem))
                for c in copies:
                    c.wait()
                ct0 = p * n_ct
                for s in range(8):
                    @pl.when(fs == s)
                    def _copy_out(s=s):
                        pltpu.sync_copy(
                            res_v,
                            out_hbm.at[h, fr, pl.ds(ct0, n_ct), s,
                                       pl.ds(0, 128)])
                plsc.subcore_barrier()
                return carry

            lax.fori_loop(0, _N_HEADS, head_body, 0)
            return carry0

        lax.fori_loop(0, 2, pass_body, 0)

    return emb


def kernel(input_ids, table):
    batch, n_heads = input_ids.shape
    rows = table.shape[0]
    tab3 = table.T.reshape(_DIM // 8, 8, rows)
    ids_flat = input_ids.T.reshape(n_heads * batch)
    tail_flat = table[rows - _TAIL:].T.reshape(_DIM * _TAIL)
    iota_flat = jnp.arange(_CBUF, dtype=jnp.int32)
    out5 = _make_kernel(batch)(tab3, ids_flat, tail_flat, iota_flat)
    out = out5.transpose(0, 1, 3, 2, 4).reshape(_N_HEADS, _DIM, batch)
    return out.transpose(2, 0, 1)


# speedup vs baseline: 2.1626x; 2.1626x over previous
import functools

import jax
import jax.numpy as jnp
from jax import lax
from jax.experimental import pallas as pl
from jax.experimental.pallas import tpu as pltpu
from jax.experimental.pallas import tpu_sc as plsc

_N_HEADS = 26
_DIM = 32
_HEAD_ROWS = 100000
_WIN = 100096
_HWIN = _WIN // 2
_TAIL = 64
_CHUNK = 128


def _make_kernel(batch):
    mesh = plsc.VectorSubcoreMesh(core_axis_name="c", subcore_axis_name="s")
    half = batch // 2
    n_ct = half // _CHUNK

    @functools.partial(
        pl.kernel,
        mesh=mesh,
        compiler_params=pltpu.CompilerParams(use_tc_tiling_on_sc=True),
        out_type=jax.ShapeDtypeStruct(
            (_N_HEADS, _DIM // 8, batch // 128, 8, 128), jnp.float32),
        scratch_types=[
            pltpu.VMEM_SHARED((8 * (_WIN + _TAIL),), jnp.float32),
            pltpu.VMEM((_HWIN + _TAIL,), jnp.float32),
            pltpu.VMEM((half,), jnp.int32),
            pltpu.VMEM((n_ct, _CHUNK), jnp.float32),
            pltpu.SemaphoreType.DMA,
        ],
    )
    def emb(tab_hbm, ids_hbm, tail_hbm, out_hbm, win_sh, stage_v, ids_v,
            res_v, sem):
        core = lax.axis_index("c")
        sub = lax.axis_index("s")
        floc = sub // 2
        p = sub % 2
        fs = floc % 8
        wbase = floc * (_WIN + _TAIL)

        def pass_body(pa, carry0):
            f = core * 16 + pa * 8 + floc
            fr = core * 2 + pa

            pltpu.sync_copy(
                tail_hbm.at[pl.ds(f * _TAIL, _TAIL)],
                stage_v.at[pl.ds(_HWIN, _TAIL)])

            def head_body(h, carry):
                off_h = h * _HEAD_ROWS
                c0 = pl.multiple_of(
                    (off_h & ~127) - (h // (_N_HEADS - 1)) * 128, 128)
                shift = off_h - c0

                cc = pl.multiple_of(c0 + p * _HWIN, 128)
                for s in range(8):
                    @pl.when(fs == s)
                    def _copy_win(s=s):
                        pltpu.sync_copy(
                            tab_hbm.at[fr, s, pl.ds(cc, _HWIN)],
                            stage_v.at[pl.ds(0, _HWIN)])
                sz = _HWIN + p * _TAIL
                pltpu.sync_copy(
                    stage_v.at[pl.ds(0, sz)],
                    win_sh.at[pl.ds(wbase + p * _HWIN, sz)])

                plsc.subcore_barrier()

                win_view = win_sh.at[pl.ds(wbase + shift, _HEAD_ROWS)]
                b0 = p * half
                pltpu.sync_copy(ids_hbm.at[pl.ds(h * batch + b0, half)],
                                ids_v)
                copies = []
                for j in range(n_ct):
                    copies.append(pltpu.async_copy(
                        win_view.at[ids_v.at[pl.ds(j * _CHUNK, _CHUNK)]],
                        res_v.at[j], sem))
                for c in copies:
                    c.wait()
                ct0 = p * n_ct
                for s in range(8):
                    @pl.when(fs == s)
                    def _copy_out(s=s):
                        pltpu.sync_copy(
                            res_v,
                            out_hbm.at[h, fr, pl.ds(ct0, n_ct), s,
                                       pl.ds(0, 128)])
                plsc.subcore_barrier()
                return carry

            lax.fori_loop(0, _N_HEADS, head_body, 0)
            return carry0

        lax.fori_loop(0, 2, pass_body, 0)

    return emb


def kernel(input_ids, table):
    batch, n_heads = input_ids.shape
    rows = table.shape[0]
    tab3 = table.T.reshape(_DIM // 8, 8, rows)
    ids_flat = input_ids.T.reshape(n_heads * batch)
    tail_flat = table[rows - _TAIL:].T.reshape(_DIM * _TAIL)
    out5 = _make_kernel(batch)(tab3, ids_flat, tail_flat)
    out = out5.transpose(0, 1, 3, 2, 4).reshape(_N_HEADS, _DIM, batch)
    return out.transpose(2, 0, 1)
